# baseline (speedup 1.0000x reference)
import functools

import jax
import jax.numpy as jnp
from jax import lax
from jax.experimental import pallas as pl
from jax.experimental.pallas import tpu as pltpu
from jax.experimental.pallas import tpu_sc as plsc

_B, _C, _H, _W = 16, 3, 512, 512
_CI = _B * _C
_NPC = _H * _W
_NBINS = 25
_HPAD = 32

_NC = 2
_NS = 16
_L = 16
_NW = _NC * _NS
_PER_W = _NPC // _NW
_NV = _PER_W // _L

_HIST_WORDS = _L * _HPAD

_HALF = _B // 4


def _sc_histograms(y3, b0, nb):
    nci = nb * _C
    hist_all = nci * _HIST_WORDS
    mesh = plsc.VectorSubcoreMesh(core_axis_name="c", subcore_axis_name="s",
                                  num_cores=_NC, num_subcores=_NS)

    @functools.partial(
        pl.kernel,
        out_type=jax.ShapeDtypeStruct((_NW * hist_all,), jnp.float32),
        mesh=mesh,
        scratch_types=[
            pltpu.VMEM((_L, _W), jnp.float32),
            pltpu.VMEM((_L, _W), jnp.float32),
            pltpu.VMEM((hist_all,), jnp.float32),
            pltpu.SemaphoreType.DMA,
            pltpu.SemaphoreType.DMA,
        ],
        compiler_params=pltpu.CompilerParams(needs_layout_passes=False,
                                             use_tc_tiling_on_sc=True),
    )
    def hist_kernel(y_hbm, out_hbm, buf0, buf1, hist, sem_a, sem_b):
        wid = lax.axis_index("s") * _NC + lax.axis_index("c")
        lane_base = lax.iota(jnp.int32, _L) * _HPAD
        ones = jnp.ones((_L,), jnp.float32)
        zeros = jnp.zeros((_L,), jnp.float32)

        @plsc.parallel_loop(0, hist_all // _L, unroll=8)
        def _(i):
            hist[pl.ds(i * _L, _L)] = zeros

        row0 = wid * _L
        ci0 = b0 * _C
        bufs = (buf0, buf1)
        sems = (sem_a, sem_b)
        copies = [None, None]
        copies[0] = pltpu.async_copy(
            y_hbm.at[ci0, pl.ds(row0, _L), :], buf0, sem_a)

        for ci in range(nci):
            slot = ci & 1
            if ci + 1 < nci:
                copies[1 - slot] = pltpu.async_copy(
                    y_hbm.at[ci0 + ci + 1, pl.ds(row0, _L), :],
                    bufs[1 - slot], sems[1 - slot])
            copies[slot].wait()
            src = bufs[slot]
            base_v = lane_base + ci * _HIST_WORDS

            @plsc.parallel_loop(0, _NV, unroll=8)
            def _(i):
                r = i >> 5
                c = i & 31
                v = src[r, pl.ds(c * _L, _L)]
                bins = (v * 25.0).astype(jnp.int32)
                plsc.addupdate_scatter(hist, [base_v + bins], ones)

        per_b = _C * _HIST_WORDS
        flushes = [
            pltpu.async_copy(
                hist.at[pl.ds(b * per_b, per_b)],
                out_hbm.at[pl.ds(b * (_NW * per_b) + wid * per_b, per_b)],
                sem_a)
            for b in range(nb)
        ]
        for f in flushes:
            f.wait()

    return hist_kernel(y3)


def _center(h, mask):
    hm = jnp.where(mask, h, 0.0)
    s = jnp.sum(hm)
    return jnp.where(mask, hm - s * (1.0 / _NBINS), 0.0)


def _tc_loss_sums(x, y, hist, b0, nb):
    def body(x_ref, y_ref, h_ref, sd_ref, sq_ref):
        b = pl.program_id(0)

        h3 = h_ref[0]
        h2 = jnp.sum(h3, axis=0)
        hr = jnp.sum(h2[0:_L], axis=0, keepdims=True)
        hg = jnp.sum(h2[_L:2 * _L], axis=0, keepdims=True)
        hb = jnp.sum(h2[2 * _L:3 * _L], axis=0, keepdims=True)

        mask = lax.broadcasted_iota(jnp.int32, (1, _HPAD), 1) < _NBINS
        ar = _center(hr, mask)
        ag = _center(hg, mask)
        ab = _center(hb, mask)
        nr = jnp.sqrt(jnp.sum(ar * ar))
        ng = jnp.sqrt(jnp.sum(ag * ag))
        nb_ = jnp.sqrt(jnp.sum(ab * ab))
        crg = jnp.sum(ar * ag) / (nr * ng)
        crb = jnp.sum(ar * ab) / (nr * nb_)
        cgb = jnp.sum(ag * ab) / (ng * nb_)

        d = x_ref[0] - y_ref[0]
        sum_d = jnp.sum(jnp.abs(d))
        d0 = d[0]
        d1 = d[1]
        d2 = d[2]
        q0 = d0 + crg * d1 + crb * d2
        q1 = crg * d0 + d1 + cgb * d2
        q2 = crb * d0 + cgb * d1 + d2
        sum_q = (jnp.sum(jnp.abs(q0)) + jnp.sum(jnp.abs(q1))
                 + jnp.sum(jnp.abs(q2)))

        @pl.when(b == 0)
        def _():
            sd_ref[0, 0] = 0.0
            sq_ref[0, 0] = 0.0

        sd_ref[0, 0] += sum_d
        sq_ref[0, 0] += sum_q

    out = pl.pallas_call(
        body,
        grid=(nb,),
        in_specs=[
            pl.BlockSpec((1, _C, _H, _W), lambda b: (b + b0, 0, 0, 0)),
            pl.BlockSpec((1, _C, _H, _W), lambda b: (b + b0, 0, 0, 0)),
            pl.BlockSpec((1, _NW, _C * _L, _HPAD), lambda b: (b, 0, 0, 0)),
        ],
        out_specs=[
            pl.BlockSpec((1, 1), lambda b: (0, 0),
                         memory_space=pltpu.SMEM),
            pl.BlockSpec((1, 1), lambda b: (0, 0),
                         memory_space=pltpu.SMEM),
        ],
        out_shape=[
            jax.ShapeDtypeStruct((1, 1), jnp.float32),
            jax.ShapeDtypeStruct((1, 1), jnp.float32),
        ],
    )(x, y, hist)
    return out[0][0, 0], out[1][0, 0]


@jax.jit
def kernel(x, y):
    y3 = y.reshape(_CI, _H, _W)
    sum_d = jnp.float32(0.0)
    sum_q = jnp.float32(0.0)
    hists = []
    for b0 in range(0, _B, _HALF):
        hf = _sc_histograms(y3, b0, _HALF)
        hists.append(hf.reshape(_HALF, _NW, _C * _L, _HPAD))
    for i, b0 in enumerate(range(0, _B, _HALF)):
        sd, sq = _tc_loss_sums(x, y, hists[i], b0, _HALF)
        sum_d = sum_d + sd
        sum_q = sum_q + sq
    n = jnp.float32(_B * _C * _H * _W)
    alpha = jnp.float32(0.8)
    beta = jnp.float32(0.2)
    return alpha * (sum_d / n) + beta * (sum_q / n)


# speedup vs baseline: 1.0294x; 1.0294x over previous
import functools

import jax
import jax.numpy as jnp
from jax import lax
from jax.experimental import pallas as pl
from jax.experimental.pallas import tpu as pltpu
from jax.experimental.pallas import tpu_sc as plsc

_B, _C, _H, _W = 16, 3, 512, 512
_CI = _B * _C
_NPC = _H * _W
_NBINS = 25
_HPAD = 32

_NC = 2
_NS = 16
_L = 16
_NW = _NC * _NS
_PER_W = _NPC // _NW
_NV = _PER_W // _L

_HIST_WORDS = _L * _HPAD

_HALF = _B // 2


def _sc_histograms(y3, b0, nb):
    nci = nb * _C
    hist_all = nci * _HIST_WORDS
    mesh = plsc.VectorSubcoreMesh(core_axis_name="c", subcore_axis_name="s",
                                  num_cores=_NC, num_subcores=_NS)

    @functools.partial(
        pl.kernel,
        out_type=jax.ShapeDtypeStruct((_NW * hist_all,), jnp.float32),
        mesh=mesh,
        scratch_types=[
            pltpu.VMEM((_L, _W), jnp.float32),
            pltpu.VMEM((_L, _W), jnp.float32),
            pltpu.VMEM((hist_all,), jnp.float32),
            pltpu.SemaphoreType.DMA,
            pltpu.SemaphoreType.DMA,
        ],
        compiler_params=pltpu.CompilerParams(needs_layout_passes=False,
                                             use_tc_tiling_on_sc=True),
    )
    def hist_kernel(y_hbm, out_hbm, buf0, buf1, hist, sem_a, sem_b):
        wid = lax.axis_index("s") * _NC + lax.axis_index("c")
        lane_base = lax.iota(jnp.int32, _L) * _HPAD
        ones = jnp.ones((_L,), jnp.float32)
        zeros = jnp.zeros((_L,), jnp.float32)

        @plsc.parallel_loop(0, hist_all // _L, unroll=8)
        def _(i):
            hist[pl.ds(i * _L, _L)] = zeros

        row0 = wid * _L
        ci0 = b0 * _C
        bufs = (buf0, buf1)
        sems = (sem_a, sem_b)
        copies = [None, None]
        copies[0] = pltpu.async_copy(
            y_hbm.at[ci0, pl.ds(row0, _L), :], buf0, sem_a)

        for ci in range(nci):
            slot = ci & 1
            if ci + 1 < nci:
                copies[1 - slot] = pltpu.async_copy(
                    y_hbm.at[ci0 + ci + 1, pl.ds(row0, _L), :],
                    bufs[1 - slot], sems[1 - slot])
            copies[slot].wait()
            src = bufs[slot]
            base_v = lane_base + ci * _HIST_WORDS

            @plsc.parallel_loop(0, _NV, unroll=8)
            def _(i):
                r = i >> 5
                c = i & 31
                v = src[r, pl.ds(c * _L, _L)]
                bins = (v * 25.0).astype(jnp.int32)
                plsc.addupdate_scatter(hist, [base_v + bins], ones)

        per_b = _C * _HIST_WORDS
        flushes = [
            pltpu.async_copy(
                hist.at[pl.ds(b * per_b, per_b)],
                out_hbm.at[pl.ds(b * (_NW * per_b) + wid * per_b, per_b)],
                sem_a)
            for b in range(nb)
        ]
        for f in flushes:
            f.wait()

    return hist_kernel(y3)


def _center(h, mask):
    hm = jnp.where(mask, h, 0.0)
    s = jnp.sum(hm)
    return jnp.where(mask, hm - s * (1.0 / _NBINS), 0.0)


def _tc_loss_sums(x, y, hist, b0, nb):
    def body(x_ref, y_ref, h_ref, sd_ref, sq_ref):
        b = pl.program_id(0)

        h3 = h_ref[0]
        h2 = jnp.sum(h3, axis=0)
        hr = jnp.sum(h2[0:_L], axis=0, keepdims=True)
        hg = jnp.sum(h2[_L:2 * _L], axis=0, keepdims=True)
        hb = jnp.sum(h2[2 * _L:3 * _L], axis=0, keepdims=True)

        mask = lax.broadcasted_iota(jnp.int32, (1, _HPAD), 1) < _NBINS
        ar = _center(hr, mask)
        ag = _center(hg, mask)
        ab = _center(hb, mask)
        nr = jnp.sqrt(jnp.sum(ar * ar))
        ng = jnp.sqrt(jnp.sum(ag * ag))
        nb_ = jnp.sqrt(jnp.sum(ab * ab))
        crg = jnp.sum(ar * ag) / (nr * ng)
        crb = jnp.sum(ar * ab) / (nr * nb_)
        cgb = jnp.sum(ag * ab) / (ng * nb_)

        d = x_ref[0] - y_ref[0]
        sum_d = jnp.sum(jnp.abs(d))
        d0 = d[0]
        d1 = d[1]
        d2 = d[2]
        q0 = d0 + crg * d1 + crb * d2
        q1 = crg * d0 + d1 + cgb * d2
        q2 = crb * d0 + cgb * d1 + d2
        sum_q = (jnp.sum(jnp.abs(q0)) + jnp.sum(jnp.abs(q1))
                 + jnp.sum(jnp.abs(q2)))

        @pl.when(b == 0)
        def _():
            sd_ref[0, 0] = 0.0
            sq_ref[0, 0] = 0.0

        sd_ref[0, 0] += sum_d
        sq_ref[0, 0] += sum_q

    out = pl.pallas_call(
        body,
        grid=(nb,),
        in_specs=[
            pl.BlockSpec((1, _C, _H, _W), lambda b: (b + b0, 0, 0, 0)),
            pl.BlockSpec((1, _C, _H, _W), lambda b: (b + b0, 0, 0, 0)),
            pl.BlockSpec((1, _NW, _C * _L, _HPAD), lambda b: (b, 0, 0, 0)),
        ],
        out_specs=[
            pl.BlockSpec((1, 1), lambda b: (0, 0),
                         memory_space=pltpu.SMEM),
            pl.BlockSpec((1, 1), lambda b: (0, 0),
                         memory_space=pltpu.SMEM),
        ],
        out_shape=[
            jax.ShapeDtypeStruct((1, 1), jnp.float32),
            jax.ShapeDtypeStruct((1, 1), jnp.float32),
        ],
    )(x, y, hist)
    return out[0][0, 0], out[1][0, 0]


@jax.jit
def kernel(x, y):
    y3 = y.reshape(_CI, _H, _W)
    sum_d = jnp.float32(0.0)
    sum_q = jnp.float32(0.0)
    hists = []
    for b0 in range(0, _B, _HALF):
        hf = _sc_histograms(y3, b0, _HALF)
        hists.append(hf.reshape(_HALF, _NW, _C * _L, _HPAD))
    for i, b0 in enumerate(range(0, _B, _HALF)):
        sd, sq = _tc_loss_sums(x, y, hists[i], b0, _HALF)
        sum_d = sum_d + sd
        sum_q = sum_q + sq
    n = jnp.float32(_B * _C * _H * _W)
    alpha = jnp.float32(0.8)
    beta = jnp.float32(0.2)
    return alpha * (sum_d / n) + beta * (sum_q / n)
